# baseline (speedup 1.0000x reference)
import functools

import jax
import jax.numpy as jnp
from jax import lax
from jax.experimental import pallas as pl
from jax.experimental.pallas import tpu as pltpu

R = 256
C = 8192
NBUF = 4


def _nce_body(nbi, njf, vt, v_total, t_ref, x_hbm, out_ref,
              buf, tbuf, m_s, s_s, g_s, sems, tsem):
    i = pl.program_id(0)
    row0 = i * R

    def start_full(row, jj, slot):
        pltpu.make_async_copy(
            x_hbm.at[pl.ds(row, R), pl.ds(jj * C, C)],
            buf.at[slot], sems.at[slot]).start()

    def start_tail(row):
        pltpu.make_async_copy(
            x_hbm.at[pl.ds(row, R), pl.ds(njf * C, vt)],
            tbuf, tsem).start()

    @pl.when(i == 0)
    def _prime():
        for k in range(NBUF):
            start_full(0, k, k)

    t = t_ref[...]

    def _accum(jj_col0, x, w):
        cols = jj_col0 + lax.broadcasted_iota(jnp.int32, (R, w), 1)
        bm = jnp.max(x, axis=1, keepdims=True)
        bs = jnp.sum(jnp.exp(x - bm), axis=1, keepdims=True)
        bg = jnp.sum(jnp.where(cols == t, x, 0.0), axis=1, keepdims=True)
        return bm, bs, bg

    def _update(jj, bm, bs, bg):
        @pl.when(jj == 0)
        def _init():
            m_s[...] = bm
            s_s[...] = bs
            g_s[...] = bg

        @pl.when(jj > 0)
        def _upd():
            m = m_s[...]
            new_m = jnp.maximum(m, bm)
            s_s[...] = s_s[...] * jnp.exp(m - new_m) + bs * jnp.exp(bm - new_m)
            m_s[...] = new_m
            g_s[...] = g_s[...] + bg

    def _step(jj, _):
        slot = lax.rem(jj, NBUF)
        pltpu.make_async_copy(
            x_hbm.at[pl.ds(row0, R), pl.ds(jj * C, C)],
            buf.at[slot], sems.at[slot]).wait()
        x = buf[slot]
        bm, bs, bg = _accum(jj * C, x, C)
        _update(jj, bm, bs, bg)

        nxt = jj + NBUF

        @pl.when(nxt < njf)
        def _sf():
            start_full(row0, nxt, lax.rem(nxt, NBUF))

        @pl.when(nxt == njf)
        def _st():
            start_tail(row0)

        @pl.when(jnp.logical_and(nxt > njf, i + 1 < nbi))
        def _sn():
            start_full(row0 + R, nxt - njf - 1, lax.rem(nxt - njf - 1, NBUF))

        return 0

    lax.fori_loop(0, njf, _step, 0)

    pltpu.make_async_copy(
        x_hbm.at[pl.ds(row0, R), pl.ds(njf * C, vt)], tbuf, tsem).wait()
    bm, bs, bg = _accum(njf * C, tbuf[...], vt)
    _update(njf, bm, bs, bg)

    @pl.when(i + 1 < nbi)
    def _sn3():
        start_full(row0 + R, NBUF - 1, NBUF - 1)

    out_ref[...] = m_s[...] + jnp.log(s_s[...]) - g_s[...]


def kernel(target, scores):
    n, v = scores.shape
    tgt = target.reshape(n, 1).astype(jnp.int32)
    nbi = n // R
    njf = v // C
    vt = v - njf * C

    loss_rows = pl.pallas_call(
        functools.partial(_nce_body, nbi, njf, vt, v),
        grid=(nbi,),
        in_specs=[
            pl.BlockSpec((R, 1), lambda i: (i, 0)),
            pl.BlockSpec(memory_space=pl.ANY),
        ],
        out_specs=pl.BlockSpec((R, 1), lambda i: (i, 0)),
        out_shape=jax.ShapeDtypeStruct((n, 1), jnp.float32),
        scratch_shapes=[
            pltpu.VMEM((NBUF, R, C), jnp.float32),
            pltpu.VMEM((R, vt), jnp.float32),
            pltpu.VMEM((R, 1), jnp.float32),
            pltpu.VMEM((R, 1), jnp.float32),
            pltpu.VMEM((R, 1), jnp.float32),
            pltpu.SemaphoreType.DMA((NBUF,)),
            pltpu.SemaphoreType.DMA,
        ],
    )(tgt, scores)

    return jnp.mean(loss_rows)
